# baseline (speedup 1.0000x reference)
import functools

import jax
import jax.numpy as jnp
from jax import lax
from jax.experimental import pallas as pl
from jax.experimental.pallas import tpu as pltpu
from jax.experimental.pallas import tpu_sc as plsc

N_ROWS = 16384
N_COLS = 1000
N_OUT = 200

NC = 2
NS = 16
NW = NC * NS

W = N_ROWS // NW
CW = 128
NCH = W // CW
SPLIT = 104


def _make_sc_kernel():
    mesh = plsc.VectorSubcoreMesh(core_axis_name="c", subcore_axis_name="s")

    @functools.partial(
        pl.kernel,
        mesh=mesh,
        out_type=jax.ShapeDtypeStruct((N_OUT, N_ROWS), jnp.float32),
        scratch_types=[
            pltpu.VMEM((N_OUT,), jnp.int32),
            pltpu.VMEM((N_OUT, CW), jnp.float32),
            pltpu.VMEM((N_OUT, CW), jnp.float32),
            pltpu.VMEM((N_OUT, CW), jnp.float32),
            pltpu.VMEM((N_OUT, CW), jnp.float32),
            pltpu.SemaphoreType.DMA,
            pltpu.SemaphoreType.DMA,
            pltpu.SemaphoreType.DMA,
            pltpu.SemaphoreType.DMA,
            pltpu.SemaphoreType.DMA,
        ],
        compiler_params=pltpu.CompilerParams(
            needs_layout_passes=False,
            disable_bounds_checks=True,
            disable_semaphore_checks=True,
            skip_device_barrier=True,
        ),
    )
    def sc_gather(xt_hbm, mask_hbm, outt_hbm, mask_v, buf0, buf1, buf2, buf3,
                  gsem0, gsem1, gsem2, gsem3, wsem):
        bufs = (buf0, buf1, buf2, buf3)
        gsems = (gsem0, gsem1, gsem2, gsem3)
        wid = lax.axis_index("s") * NC + lax.axis_index("c")
        c0 = wid * W

        pltpu.sync_copy(mask_hbm, mask_v)

        def gathers(c):
            col = c0 + c * CW
            return (
                pltpu.make_async_copy(
                    xt_hbm.at[mask_v.at[pl.ds(0, SPLIT)], pl.ds(col, CW)],
                    bufs[c].at[pl.ds(0, SPLIT), :], gsems[c],
                ),
                pltpu.make_async_copy(
                    xt_hbm.at[
                        mask_v.at[pl.ds(SPLIT, N_OUT - SPLIT)],
                        pl.ds(col, CW),
                    ],
                    bufs[c].at[pl.ds(SPLIT, N_OUT - SPLIT), :],
                    gsems[c],
                ),
            )

        def writeback(c):
            col = c0 + c * CW
            return pltpu.make_async_copy(
                bufs[c], outt_hbm.at[:, pl.ds(col, CW)], wsem
            )

        for c in range(NCH):
            for g in gathers(c):
                g.start()
        for c in range(NCH):
            for g in gathers(c):
                g.wait()
            writeback(c).start()
        for c in range(NCH):
            writeback(c).wait()

    return sc_gather


_sc_gather = _make_sc_kernel()


@jax.jit
def kernel(x, mask):
    out_t = _sc_gather(x.T, mask.astype(jnp.int32))
    return out_t.T


# speedup vs baseline: 1.1611x; 1.1611x over previous
import functools

import jax
import jax.numpy as jnp
from jax import lax
from jax.experimental import pallas as pl
from jax.experimental.pallas import tpu as pltpu
from jax.experimental.pallas import tpu_sc as plsc

N_ROWS = 16384
N_COLS = 1000
N_OUT = 200

NC = 2
NS = 16
NW = NC * NS

W = N_ROWS // NW
CW = 256
NCH = W // CW
SPLIT = 104


def _make_sc_kernel():
    mesh = plsc.VectorSubcoreMesh(core_axis_name="c", subcore_axis_name="s")

    @functools.partial(
        pl.kernel,
        mesh=mesh,
        out_type=jax.ShapeDtypeStruct((N_OUT, N_ROWS), jnp.float32),
        scratch_types=[
            pltpu.VMEM((N_OUT,), jnp.int32),
            pltpu.VMEM((N_OUT, CW), jnp.float32),
            pltpu.VMEM((N_OUT, CW), jnp.float32),
            pltpu.SemaphoreType.DMA,
            pltpu.SemaphoreType.DMA,
            pltpu.SemaphoreType.DMA,
        ],
        compiler_params=pltpu.CompilerParams(
            needs_layout_passes=False,
            disable_bounds_checks=True,
            disable_semaphore_checks=True,
            skip_device_barrier=True,
        ),
    )
    def sc_gather(xt_hbm, mask_hbm, outt_hbm, mask_v, buf0, buf1,
                  gsem0, gsem1, wsem):
        bufs = (buf0, buf1)
        gsems = (gsem0, gsem1)
        wid = lax.axis_index("s") * NC + lax.axis_index("c")
        c0 = wid * W

        pltpu.sync_copy(mask_hbm, mask_v)

        def gathers(c):
            col = c0 + c * CW
            return (
                pltpu.make_async_copy(
                    xt_hbm.at[mask_v.at[pl.ds(0, SPLIT)], pl.ds(col, CW)],
                    bufs[c].at[pl.ds(0, SPLIT), :], gsems[c],
                ),
                pltpu.make_async_copy(
                    xt_hbm.at[
                        mask_v.at[pl.ds(SPLIT, N_OUT - SPLIT)],
                        pl.ds(col, CW),
                    ],
                    bufs[c].at[pl.ds(SPLIT, N_OUT - SPLIT), :],
                    gsems[c],
                ),
            )

        def writeback(c):
            col = c0 + c * CW
            return pltpu.make_async_copy(
                bufs[c], outt_hbm.at[:, pl.ds(col, CW)], wsem
            )

        for c in range(NCH):
            for g in gathers(c):
                g.start()
        for c in range(NCH):
            for g in gathers(c):
                g.wait()
            writeback(c).start()
        for c in range(NCH):
            writeback(c).wait()

    return sc_gather


_sc_gather = _make_sc_kernel()


@jax.jit
def kernel(x, mask):
    out_t = _sc_gather(x.T, mask.astype(jnp.int32))
    return out_t.T


# speedup vs baseline: 1.1784x; 1.0149x over previous
import functools

import jax
import jax.numpy as jnp
from jax import lax
from jax.experimental import pallas as pl
from jax.experimental.pallas import tpu as pltpu
from jax.experimental.pallas import tpu_sc as plsc

N_ROWS = 16384
N_COLS = 1000
N_OUT = 200

NC = 2
NS = 16
NW = NC * NS

W = N_ROWS // NW
SPLIT = 104


def _make_sc_kernel():
    mesh = plsc.VectorSubcoreMesh(core_axis_name="c", subcore_axis_name="s")

    @functools.partial(
        pl.kernel,
        mesh=mesh,
        out_type=jax.ShapeDtypeStruct((N_OUT, N_ROWS), jnp.float32),
        scratch_types=[
            pltpu.VMEM((N_OUT,), jnp.int32),
            pltpu.VMEM((SPLIT, W), jnp.float32),
            pltpu.VMEM((N_OUT - SPLIT, W), jnp.float32),
            pltpu.SemaphoreType.DMA,
            pltpu.SemaphoreType.DMA,
            pltpu.SemaphoreType.DMA,
            pltpu.SemaphoreType.DMA,
        ],
        compiler_params=pltpu.CompilerParams(
            needs_layout_passes=False,
            disable_bounds_checks=True,
            disable_semaphore_checks=True,
            skip_device_barrier=True,
        ),
    )
    def sc_gather(xt_hbm, mask_hbm, outt_hbm, mask_v, buf0, buf1,
                  gsem0, gsem1, wsem0, wsem1):
        wid = lax.axis_index("s") * NC + lax.axis_index("c")
        c0 = wid * W

        pltpu.sync_copy(mask_hbm, mask_v)

        g0 = pltpu.make_async_copy(
            xt_hbm.at[mask_v.at[pl.ds(0, SPLIT)], pl.ds(c0, W)], buf0, gsem0
        )
        g1 = pltpu.make_async_copy(
            xt_hbm.at[mask_v.at[pl.ds(SPLIT, N_OUT - SPLIT)], pl.ds(c0, W)],
            buf1, gsem1,
        )
        w0 = pltpu.make_async_copy(
            buf0, outt_hbm.at[pl.ds(0, SPLIT), pl.ds(c0, W)], wsem0
        )
        w1 = pltpu.make_async_copy(
            buf1, outt_hbm.at[pl.ds(SPLIT, N_OUT - SPLIT), pl.ds(c0, W)], wsem1
        )

        g0.start()
        g1.start()
        g0.wait()
        w0.start()
        g1.wait()
        w1.start()
        w0.wait()
        w1.wait()

    return sc_gather


_sc_gather = _make_sc_kernel()


@jax.jit
def kernel(x, mask):
    out_t = _sc_gather(x.T, mask.astype(jnp.int32))
    return out_t.T
